# baseline (speedup 1.0000x reference)
import functools

import jax
import jax.numpy as jnp
from jax import lax
from jax.experimental import pallas as pl
from jax.experimental.pallas import tpu as pltpu
from jax.experimental.pallas import tpu_sc as plsc

_NUM_OPT = 4
_NUM_D = 16
_LANES = 16
_NC = 2
_NS = 16
_NW = _NC * _NS
_BATCH = 16384
_RPW = _BATCH // _NW
_CHUNK = 128
_NCHUNK = _RPW // _CHUNK
_ACOL = _NUM_D
_BCOL = _NUM_D + _NUM_OPT * _NUM_D


def _irt_body(uid_hbm, uid8_hbm, qid_hbm, tabth_hbm, tabab_hbm, out_hbm,
              uid_v, uid8_v, qid_v, u_b0, u_b1, q_b0, q_b1, o_v, sem0, sem1):
    wid = lax.axis_index("s") * _NC + lax.axis_index("c")

    pltpu.sync_copy(uid_hbm.at[wid], uid_v)
    pltpu.sync_copy(uid8_hbm.at[wid], uid8_v)
    pltpu.sync_copy(qid_hbm.at[wid], qid_v)

    u_b = (u_b0, u_b1)
    q_b = (q_b0, q_b1)
    sems = (sem0, sem1)

    def fire(j):
        p = j % 2
        return [
            pltpu.async_copy(tabth_hbm.at[uid8_v.at[j]], u_b[p], sems[p]),
            pltpu.async_copy(tabab_hbm.at[qid_v.at[j]], q_b[p], sems[p]),
        ]

    lanes = lax.iota(jnp.int32, _LANES)

    def compute(j):
        p = j % 2
        for blk in range(_CHUNK // _LANES):
            rloc = lanes + blk * _LANES
            uv = plsc.load_gather(uid_v, [jnp.full((_LANES,), j, jnp.int32), rloc])
            thbase = lax.shift_left(lax.bitwise_and(uv, 7), 4)
            acc = [plsc.load_gather(q_b[p], [rloc, jnp.full((_LANES,), 64 + k, jnp.int32)])
                   for k in range(_NUM_OPT)]
            for d in range(_NUM_D):
                th_d = plsc.load_gather(u_b[p], [rloc, thbase + d])
                for k in range(_NUM_OPT):
                    a_kd = plsc.load_gather(
                        q_b[p], [rloc, jnp.full((_LANES,), k * _NUM_D + d, jnp.int32)])
                    acc[k] = acc[k] + a_kd * th_d
            for k in range(_NUM_OPT):
                flat = (rloc + j * _CHUNK) * _NUM_OPT + k
                plsc.store_scatter(
                    o_v,
                    [lax.shift_right_logical(flat, 7), lax.bitwise_and(flat, 127)],
                    acc[k])

    pending = fire(0)
    for j in range(_NCHUNK):
        nxt = fire(j + 1) if j + 1 < _NCHUNK else []
        for c in pending:
            c.wait()
        pending = nxt
        compute(j)

    nrow_o = _RPW * _NUM_OPT // 128
    pltpu.sync_copy(o_v, out_hbm.at[pl.ds(wid * nrow_o, nrow_o)])


_sc_call = functools.partial(
    pl.kernel,
    mesh=plsc.VectorSubcoreMesh(core_axis_name="c", subcore_axis_name="s"),
    compiler_params=pltpu.CompilerParams(
        needs_layout_passes=False, use_tc_tiling_on_sc=True),
    out_type=jax.ShapeDtypeStruct((_BATCH * _NUM_OPT // 128, 128), jnp.float32),
    scratch_types=[
        pltpu.VMEM((_NCHUNK, _CHUNK), jnp.int32),
        pltpu.VMEM((_NCHUNK, _CHUNK), jnp.int32),
        pltpu.VMEM((_NCHUNK, _CHUNK), jnp.int32),
        pltpu.VMEM((_CHUNK, 128), jnp.float32),
        pltpu.VMEM((_CHUNK, 128), jnp.float32),
        pltpu.VMEM((_CHUNK, 128), jnp.float32),
        pltpu.VMEM((_CHUNK, 128), jnp.float32),
        pltpu.VMEM((_RPW * _NUM_OPT // 128, 128), jnp.float32),
        pltpu.SemaphoreType.DMA,
        pltpu.SemaphoreType.DMA,
    ],
)(_irt_body)


@jax.jit
def kernel(x, a, b, theta):
    uids = x[:, 0].astype(jnp.int32).reshape(_NW, _NCHUNK, _CHUNK)
    qids = x[:, 1].astype(jnp.int32).reshape(_NW, _NCHUNK, _CHUNK)
    n = theta.shape[0]
    uid8 = lax.shift_right_logical(uids, 3)
    tab_th = theta.reshape(n * _NUM_D // 128, 128)
    tab_ab = (jnp.pad(a.reshape(n, _NUM_OPT * _NUM_D), ((0, 0), (0, 64)))
              + jnp.pad(b, ((0, 0), (64, 60))))
    out = _sc_call(uids, uid8, qids, tab_th, tab_ab)
    return out.reshape(_BATCH, _NUM_OPT)


# speedup vs baseline: 1.0742x; 1.0742x over previous
import functools

import jax
import jax.numpy as jnp
from jax import lax
from jax.experimental import pallas as pl
from jax.experimental.pallas import tpu as pltpu
from jax.experimental.pallas import tpu_sc as plsc

_NUM_OPT = 4
_NUM_D = 16
_LANES = 16
_NC = 2
_NS = 16
_NW = _NC * _NS
_BATCH = 16384
_RPW = _BATCH // _NW
_CHUNK = 128
_NCHUNK = _RPW // _CHUNK
_ACOL = _NUM_D
_BCOL = _NUM_D + _NUM_OPT * _NUM_D


def _irt_body(uid_hbm, qid_hbm, tab_hbm, out_hbm,
              uid_v, qid_v, u_b0, u_b1, q_b0, q_b1, o_v, sem0, sem1):
    wid = lax.axis_index("s") * _NC + lax.axis_index("c")

    pltpu.sync_copy(uid_hbm.at[wid], uid_v)
    pltpu.sync_copy(qid_hbm.at[wid], qid_v)

    u_b = (u_b0, u_b1)
    q_b = (q_b0, q_b1)
    sems = (sem0, sem1)

    def fire(j):
        p = j % 2
        return [
            pltpu.async_copy(tab_hbm.at[uid_v.at[j]], u_b[p], sems[p]),
            pltpu.async_copy(tab_hbm.at[qid_v.at[j]], q_b[p], sems[p]),
        ]

    lanes = lax.iota(jnp.int32, _LANES)

    def compute(j):
        p = j % 2
        for blk in range(_CHUNK // _LANES):
            rloc = lanes + blk * _LANES
            acc = [plsc.load_gather(q_b[p], [rloc, jnp.full((_LANES,), _BCOL + k, jnp.int32)])
                   for k in range(_NUM_OPT)]
            for d in range(_NUM_D):
                th_d = plsc.load_gather(u_b[p], [rloc, jnp.full((_LANES,), d, jnp.int32)])
                for k in range(_NUM_OPT):
                    a_kd = plsc.load_gather(
                        q_b[p], [rloc, jnp.full((_LANES,), _ACOL + k * _NUM_D + d, jnp.int32)])
                    acc[k] = acc[k] + a_kd * th_d
            for k in range(_NUM_OPT):
                flat = (rloc + j * _CHUNK) * _NUM_OPT + k
                plsc.store_scatter(
                    o_v,
                    [lax.shift_right_logical(flat, 7), lax.bitwise_and(flat, 127)],
                    acc[k])

    pending = fire(0)
    for j in range(_NCHUNK):
        nxt = fire(j + 1) if j + 1 < _NCHUNK else []
        for c in pending:
            c.wait()
        pending = nxt
        compute(j)

    nrow_o = _RPW * _NUM_OPT // 128
    pltpu.sync_copy(o_v, out_hbm.at[pl.ds(wid * nrow_o, nrow_o)])


_sc_call = functools.partial(
    pl.kernel,
    mesh=plsc.VectorSubcoreMesh(core_axis_name="c", subcore_axis_name="s"),
    compiler_params=pltpu.CompilerParams(
        needs_layout_passes=False, use_tc_tiling_on_sc=True),
    out_type=jax.ShapeDtypeStruct((_BATCH * _NUM_OPT // 128, 128), jnp.float32),
    scratch_types=[
        pltpu.VMEM((_NCHUNK, _CHUNK), jnp.int32),
        pltpu.VMEM((_NCHUNK, _CHUNK), jnp.int32),
        pltpu.VMEM((_CHUNK, 128), jnp.float32),
        pltpu.VMEM((_CHUNK, 128), jnp.float32),
        pltpu.VMEM((_CHUNK, 128), jnp.float32),
        pltpu.VMEM((_CHUNK, 128), jnp.float32),
        pltpu.VMEM((_RPW * _NUM_OPT // 128, 128), jnp.float32),
        pltpu.SemaphoreType.DMA,
        pltpu.SemaphoreType.DMA,
    ],
)(_irt_body)


@jax.jit
def kernel(x, a, b, theta):
    uids = x[:, 0].astype(jnp.int32).reshape(_NW, _NCHUNK, _CHUNK)
    qids = x[:, 1].astype(jnp.int32).reshape(_NW, _NCHUNK, _CHUNK)
    n = theta.shape[0]
    tab = (jnp.pad(theta, ((0, 0), (0, 128 - _NUM_D)))
           + jnp.pad(a.reshape(n, _NUM_OPT * _NUM_D),
                     ((0, 0), (_ACOL, 128 - _BCOL)))
           + jnp.pad(b, ((0, 0), (_BCOL, 128 - _BCOL - _NUM_OPT))))
    out = _sc_call(uids, qids, tab)
    return out.reshape(_BATCH, _NUM_OPT)
